# baseline (speedup 1.0000x reference)
import functools

import jax
import jax.numpy as jnp
from jax import lax
from jax.experimental import pallas as pl
from jax.experimental.pallas import tpu as pltpu
from jax.experimental.shard_map import shard_map
from jax.sharding import Mesh, PartitionSpec as P

_LOG2E = 1.4426950408889634
_LN2 = 0.6931471805599453
_INV_TEMPS = (1.0, 2.0)
_TM = 1024


def _online_tile(scores_bf, m_scr, l_scr, tm):
    chunks = [scores_bf[:, g * 128:(g + 1) * 128] for g in range(tm // 128)]
    tile_m = chunks[0]
    for c in chunks[1:]:
        tile_m = jnp.maximum(tile_m, c)
    m_prev = m_scr[...]
    m_new = jnp.maximum(m_prev, tile_m)
    part = jnp.exp2(chunks[0] - m_new)
    for c in chunks[1:]:
        part = part + jnp.exp2(c - m_new)
    alpha = jnp.exp2((m_prev - m_new).astype(jnp.float32))
    l_scr[...] = alpha * l_scr[...] + part.astype(jnp.float32)
    m_scr[...] = m_new


def _final_stats(m_scr, l_scr, pos_dist, n):
    m_f32 = m_scr[...].astype(jnp.float32)
    m_row = jnp.max(m_f32, axis=-1, keepdims=True)
    l_row = jnp.sum(l_scr[...] * jnp.exp2(m_f32 - m_row),
                    axis=-1, keepdims=True)
    inv_n = jnp.float32(1.0 / n)
    align = jnp.sum(m_row - pos_dist) * jnp.float32(_LN2) * inv_n
    uniform = jnp.sum(jnp.log(l_row)) * inv_n
    return align, uniform


def _single_crit_kernel(refs_ref, pos_ref, neg_ref, out_ref, m_scr, l_scr,
                        *, n, tm, num_mt, scale):
    mi = pl.program_id(0)

    @pl.when(mi == 0)
    def _init():
        m_scr[...] = jnp.full(m_scr.shape, -jnp.inf, dtype=jnp.bfloat16)
        l_scr[...] = jnp.zeros(l_scr.shape, dtype=jnp.float32)

    scores = lax.dot_general(
        refs_ref[...], neg_ref[...].astype(jnp.bfloat16),
        dimension_numbers=(((1,), (1,)), ((), ())),
        preferred_element_type=jnp.float32,
    ).astype(jnp.bfloat16)
    _online_tile(scores, m_scr, l_scr, tm)

    @pl.when(mi == num_mt - 1)
    def _finalize():
        pos_dist = jnp.sum(refs_ref[...].astype(jnp.float32)
                           * pos_ref[...].astype(jnp.float32),
                           axis=-1, keepdims=True)
        align, uniform = _final_stats(m_scr, l_scr, pos_dist, n)
        out_ref[0, 0] = align + uniform
        out_ref[0, 1] = align
        out_ref[0, 2] = uniform


def _one_criterion(refs_bf, pos_bf, neg):
    n, d = refs_bf.shape
    m = neg.shape[0]
    tm = _TM if m % _TM == 0 else m
    num_mt = m // tm
    row_spec = pl.BlockSpec((n, d), lambda mi: (0, 0))
    body = functools.partial(_single_crit_kernel, n=n, tm=tm, num_mt=num_mt,
                             scale=1.0)
    return pl.pallas_call(
        body,
        grid=(num_mt,),
        in_specs=[row_spec, row_spec,
                  pl.BlockSpec((tm, d), lambda mi: (mi, 0))],
        out_specs=pl.BlockSpec(memory_space=pltpu.MemorySpace.SMEM),
        out_shape=jax.ShapeDtypeStruct((1, 3), jnp.float32),
        scratch_shapes=[
            pltpu.VMEM((n, 128), jnp.bfloat16),
            pltpu.VMEM((n, 128), jnp.float32),
        ],
        compiler_params=pltpu.CompilerParams(
            dimension_semantics=("arbitrary",)),
    )(refs_bf, pos_bf, neg)


def _dual_crit_kernel(ref1_ref, pos1_ref, neg1_ref, ref2_ref, pos2_ref,
                      neg2_ref, out_ref, r1s, r2s, m1, l1, m2, l2,
                      *, n, tm, num_mt):
    mi = pl.program_id(0)

    @pl.when(mi == 0)
    def _init():
        r1s[...] = (ref1_ref[...] * jnp.float32(_INV_TEMPS[0] * _LOG2E)
                    ).astype(jnp.bfloat16)
        r2s[...] = (ref2_ref[...] * jnp.float32(_INV_TEMPS[1] * _LOG2E)
                    ).astype(jnp.bfloat16)
        for m_scr, l_scr in ((m1, l1), (m2, l2)):
            m_scr[...] = jnp.full(m_scr.shape, -jnp.inf, dtype=jnp.bfloat16)
            l_scr[...] = jnp.zeros(l_scr.shape, dtype=jnp.float32)

    for refs_s, neg_ref, m_scr, l_scr in ((r1s, neg1_ref, m1, l1),
                                          (r2s, neg2_ref, m2, l2)):
        scores = lax.dot_general(
            refs_s[...], neg_ref[...].astype(jnp.bfloat16),
            dimension_numbers=(((1,), (1,)), ((), ())),
            preferred_element_type=jnp.float32,
        ).astype(jnp.bfloat16)
        _online_tile(scores, m_scr, l_scr, tm)

    @pl.when(mi == num_mt - 1)
    def _finalize():
        finals = (
            (ref1_ref, pos1_ref, m1, l1, _INV_TEMPS[0] * _LOG2E, 0),
            (ref2_ref, pos2_ref, m2, l2, _INV_TEMPS[1] * _LOG2E, 1),
        )
        for ref_ref, pos_ref, m_scr, l_scr, scale, k in finals:
            pos_dist = jnp.sum(ref_ref[...] * jnp.float32(scale) * pos_ref[...],
                               axis=-1, keepdims=True)
            align, uniform = _final_stats(m_scr, l_scr, pos_dist, n)
            out_ref[k, 0] = align + uniform
            out_ref[k, 1] = align
            out_ref[k, 2] = uniform


def _dual_call(ref1, pos1, neg1, ref2, pos2, neg2):
    n, d = ref1.shape
    m = neg1.shape[0]
    tm = _TM if m % _TM == 0 else m
    num_mt = m // tm
    row_spec = pl.BlockSpec((n, d), lambda mi: (0, 0))
    neg_spec = pl.BlockSpec((tm, d), lambda mi: (mi, 0))
    body = functools.partial(_dual_crit_kernel, n=n, tm=tm, num_mt=num_mt)
    return pl.pallas_call(
        body,
        grid=(num_mt,),
        in_specs=[row_spec, row_spec, neg_spec, row_spec, row_spec, neg_spec],
        out_specs=pl.BlockSpec(memory_space=pltpu.MemorySpace.SMEM),
        out_shape=jax.ShapeDtypeStruct((2, 3), jnp.float32),
        scratch_shapes=[
            pltpu.VMEM((n, d), jnp.bfloat16),
            pltpu.VMEM((n, d), jnp.bfloat16),
            pltpu.VMEM((n, 128), jnp.bfloat16),
            pltpu.VMEM((n, 128), jnp.float32),
            pltpu.VMEM((n, 128), jnp.bfloat16),
            pltpu.VMEM((n, 128), jnp.float32),
        ],
        compiler_params=pltpu.CompilerParams(
            dimension_semantics=("arbitrary",)),
    )(ref1, pos1, neg1, ref2, pos2, neg2)


def kernel(ref1, pos1, neg1, ref2, pos2, neg2):
    tpu_devs = [d for d in jax.devices() if d.platform in ("tpu", "cpu")]
    if len(tpu_devs) >= 2:
        mesh = Mesh(tpu_devs[:2], ("c",))

        def _shard_body(r1, p1, g1, r2, p2, g2):
            idx = lax.axis_index("c")
            refs, pos, neg = lax.cond(
                idx == 0,
                lambda: (r1 * jnp.float32(_INV_TEMPS[0] * _LOG2E), p1, g1),
                lambda: (r2 * jnp.float32(_INV_TEMPS[1] * _LOG2E), p2, g2),
            )
            out = _one_criterion(refs.astype(jnp.bfloat16),
                                 pos.astype(jnp.bfloat16), neg)
            return out

        rep = P(None, None)
        return shard_map(
            _shard_body, mesh=mesh,
            in_specs=(rep,) * 6,
            out_specs=P("c", None),
            check_rep=False,
        )(ref1, pos1, neg1, ref2, pos2, neg2)
    return _dual_call(ref1, pos1, neg1, ref2, pos2, neg2)


# speedup vs baseline: 2.4002x; 2.4002x over previous
import functools

import jax
import jax.numpy as jnp
from jax import lax
from jax.experimental import pallas as pl
from jax.experimental.pallas import tpu as pltpu

_LOG2E = 1.4426950408889634
_LN2 = 0.6931471805599453
_INV_TEMPS = (1.0, 2.0)
_TM = 1024
_NEG_BIG = -1.0e30


def _infonce2_kernel(ref1_ref, pos1_ref, neg1_ref, ref2_ref, pos2_ref,
                     neg2_ref, out_ref, r1s, r2s, m1, l1, m2, l2,
                     *, n, tm, num_mt):
    mi = pl.program_id(0)

    @pl.when(mi == 0)
    def _init():
        r1s[...] = (ref1_ref[...] * jnp.float32(_INV_TEMPS[0] * _LOG2E)
                    ).astype(jnp.bfloat16)
        r2s[...] = (ref2_ref[...] * jnp.float32(_INV_TEMPS[1] * _LOG2E)
                    ).astype(jnp.bfloat16)
        for m_scr, l_scr in ((m1, l1), (m2, l2)):
            m_scr[...] = jnp.full(m_scr.shape, _NEG_BIG, dtype=jnp.bfloat16)
            l_scr[...] = jnp.zeros(l_scr.shape, dtype=jnp.float32)

    for refs_s, neg_ref, m_scr, l_scr in ((r1s, neg1_ref, m1, l1),
                                          (r2s, neg2_ref, m2, l2)):
        scores = lax.dot_general(
            neg_ref[...].astype(jnp.bfloat16), refs_s[...],
            dimension_numbers=(((1,), (1,)), ((), ())),
            preferred_element_type=jnp.float32,
        ).astype(jnp.bfloat16)
        tile_m = jnp.max(scores, axis=0, keepdims=True)
        m_prev = m_scr[...]
        m_new = jnp.maximum(m_prev, tile_m)
        part = jnp.sum(jnp.exp2(scores - m_new), axis=0, keepdims=True,
                       dtype=jnp.bfloat16)
        alpha = jnp.exp2((m_prev - m_new).astype(jnp.float32))
        l_scr[...] = alpha * l_scr[...] + part.astype(jnp.float32)
        m_scr[...] = m_new

    @pl.when(mi == num_mt - 1)
    def _finalize():
        finals = (
            (ref1_ref, pos1_ref, m1, l1, _INV_TEMPS[0] * _LOG2E, 0),
            (ref2_ref, pos2_ref, m2, l2, _INV_TEMPS[1] * _LOG2E, 1),
        )
        inv_n = jnp.float32(1.0 / n)
        for ref_ref, pos_ref, m_scr, l_scr, scale, k in finals:
            pos_sum = jnp.sum(ref_ref[...] * jnp.float32(scale) * pos_ref[...])
            m_sum = jnp.sum(m_scr[...].astype(jnp.float32))
            align = (m_sum - pos_sum) * jnp.float32(_LN2) * inv_n
            uniform = jnp.sum(jnp.log(l_scr[...])) * inv_n
            out_ref[k, 0] = align + uniform
            out_ref[k, 1] = align
            out_ref[k, 2] = uniform


def kernel(ref1, pos1, neg1, ref2, pos2, neg2):
    n, d = ref1.shape
    m = neg1.shape[0]
    tm = _TM if m % _TM == 0 else m
    num_mt = m // tm

    row_spec = pl.BlockSpec((n, d), lambda mi: (0, 0))
    neg_spec = pl.BlockSpec((tm, d), lambda mi: (mi, 0))
    body = functools.partial(_infonce2_kernel, n=n, tm=tm, num_mt=num_mt)
    return pl.pallas_call(
        body,
        grid=(num_mt,),
        in_specs=[row_spec, row_spec, neg_spec, row_spec, row_spec, neg_spec],
        out_specs=pl.BlockSpec(memory_space=pltpu.MemorySpace.SMEM),
        out_shape=jax.ShapeDtypeStruct((2, 3), jnp.float32),
        scratch_shapes=[
            pltpu.VMEM((n, d), jnp.bfloat16),
            pltpu.VMEM((n, d), jnp.bfloat16),
            pltpu.VMEM((1, n), jnp.bfloat16),
            pltpu.VMEM((1, n), jnp.float32),
            pltpu.VMEM((1, n), jnp.bfloat16),
            pltpu.VMEM((1, n), jnp.float32),
        ],
        compiler_params=pltpu.CompilerParams(
            dimension_semantics=("arbitrary",)),
    )(ref1, pos1, neg1, ref2, pos2, neg2)


# speedup vs baseline: 2.8650x; 1.1937x over previous
import functools

import jax
import jax.numpy as jnp
from jax import lax
from jax.experimental import pallas as pl
from jax.experimental.pallas import tpu as pltpu

_LOG2E = 1.4426950408889634
_LN2 = 0.6931471805599453
_INV_TEMPS = (1.0, 2.0)
_TM = 1024
_NEG_BIG = -1.0e30


def _lane_chunks(scores, tm):
    return [scores[:, g * 128:(g + 1) * 128] for g in range(tm // 128)]


def _tree_max(chunks):
    t = chunks[0]
    for c in chunks[1:]:
        t = jnp.maximum(t, c)
    return t


def _mono_kernel(ref1_ref, pos1_ref, neg1_ref, ref2_ref, pos2_ref, neg2_ref,
                 out_ref, *, n, tm, num_mt):
    crits = (
        (ref1_ref, pos1_ref, neg1_ref, _INV_TEMPS[0] * _LOG2E, 0),
        (ref2_ref, pos2_ref, neg2_ref, _INV_TEMPS[1] * _LOG2E, 1),
    )
    inv_n = jnp.float32(1.0 / n)
    for ref_ref, pos_ref, neg_ref, scale, k in crits:
        refs_s = (ref_ref[...] * jnp.float32(scale)).astype(jnp.bfloat16)
        m = jnp.full((n, 128), _NEG_BIG, dtype=jnp.bfloat16)
        l = jnp.zeros((n, 128), dtype=jnp.float32)
        for t in range(num_mt):
            scores = lax.dot_general(
                refs_s, neg_ref[t * tm:(t + 1) * tm, :].astype(jnp.bfloat16),
                dimension_numbers=(((1,), (1,)), ((), ())),
                preferred_element_type=jnp.float32,
            ).astype(jnp.bfloat16)
            chunks = _lane_chunks(scores, tm)
            m_new = jnp.maximum(m, _tree_max(chunks))
            part = jnp.exp2(chunks[0] - m_new)
            for c in chunks[1:]:
                part = part + jnp.exp2(c - m_new)
            l = jnp.exp2((m - m_new).astype(jnp.float32)) * l \
                + part.astype(jnp.float32)
            m = m_new
        m_f32 = m.astype(jnp.float32)
        m_row = jnp.max(m_f32, axis=-1, keepdims=True)
        l_row = jnp.sum(l * jnp.exp2(m_f32 - m_row), axis=-1, keepdims=True)
        pos_dist = jnp.sum(ref_ref[...] * jnp.float32(scale) * pos_ref[...],
                           axis=-1, keepdims=True)
        align = jnp.sum(m_row - pos_dist) * jnp.float32(_LN2) * inv_n
        uniform = jnp.sum(jnp.log(l_row)) * inv_n
        out_ref[k, 0] = align + uniform
        out_ref[k, 1] = align
        out_ref[k, 2] = uniform


def kernel(ref1, pos1, neg1, ref2, pos2, neg2):
    n, d = ref1.shape
    m = neg1.shape[0]
    tm = _TM if m % _TM == 0 else m
    num_mt = m // tm
    row_spec = pl.BlockSpec((n, d), lambda: (0, 0))
    neg_spec = pl.BlockSpec((m, d), lambda: (0, 0))
    body = functools.partial(_mono_kernel, n=n, tm=tm, num_mt=num_mt)
    return pl.pallas_call(
        body,
        grid=(),
        in_specs=[row_spec, row_spec, neg_spec, row_spec, row_spec, neg_spec],
        out_specs=pl.BlockSpec(memory_space=pltpu.MemorySpace.SMEM),
        out_shape=jax.ShapeDtypeStruct((2, 3), jnp.float32),
        compiler_params=pltpu.CompilerParams(),
    )(ref1, pos1, neg1, ref2, pos2, neg2)


# speedup vs baseline: 3.1702x; 1.1065x over previous
import functools

import jax
import jax.numpy as jnp
from jax import lax
from jax.experimental import pallas as pl
from jax.experimental.pallas import tpu as pltpu

_LOG2E = 1.4426950408889634
_LN2 = 0.6931471805599453
_INV_TEMPS = (1.0, 2.0)
_TM = 512
_NEG_BIG = -1.0e30


def _lane_chunks(scores, tm):
    return [scores[:, g * 128:(g + 1) * 128] for g in range(tm // 128)]


def _tree_max(chunks):
    t = chunks[0]
    for c in chunks[1:]:
        t = jnp.maximum(t, c)
    return t


def _mono_kernel(ref1_ref, pos1_ref, neg1_ref, ref2_ref, pos2_ref, neg2_ref,
                 out_ref, *, n, tm, num_mt):
    crits = (
        (ref1_ref, pos1_ref, neg1_ref, _INV_TEMPS[0] * _LOG2E, 0),
        (ref2_ref, pos2_ref, neg2_ref, _INV_TEMPS[1] * _LOG2E, 1),
    )
    inv_n = jnp.float32(1.0 / n)
    for ref_ref, pos_ref, neg_ref, scale, k in crits:
        refs_s = (ref_ref[...] * jnp.float32(scale)).astype(jnp.bfloat16)
        m = jnp.full((n, 128), _NEG_BIG, dtype=jnp.bfloat16)
        l = jnp.zeros((n, 128), dtype=jnp.float32)
        for t in range(num_mt):
            scores = lax.dot_general(
                refs_s, neg_ref[t * tm:(t + 1) * tm, :].astype(jnp.bfloat16),
                dimension_numbers=(((1,), (1,)), ((), ())),
                preferred_element_type=jnp.float32,
            ).astype(jnp.bfloat16)
            chunks = _lane_chunks(scores, tm)
            m_new = jnp.maximum(m, _tree_max(chunks))
            part = jnp.exp2(chunks[0] - m_new)
            for c in chunks[1:]:
                part = part + jnp.exp2(c - m_new)
            l = jnp.exp2((m - m_new).astype(jnp.float32)) * l \
                + part.astype(jnp.float32)
            m = m_new
        m_f32 = m.astype(jnp.float32)
        m_row = jnp.max(m_f32, axis=-1, keepdims=True)
        l_row = jnp.sum(l * jnp.exp2(m_f32 - m_row), axis=-1, keepdims=True)
        pos_dist = jnp.sum(ref_ref[...] * jnp.float32(scale) * pos_ref[...],
                           axis=-1, keepdims=True)
        align = jnp.sum(m_row - pos_dist) * jnp.float32(_LN2) * inv_n
        uniform = jnp.sum(jnp.log(l_row)) * inv_n
        out_ref[k, 0] = align + uniform
        out_ref[k, 1] = align
        out_ref[k, 2] = uniform


def kernel(ref1, pos1, neg1, ref2, pos2, neg2):
    n, d = ref1.shape
    m = neg1.shape[0]
    tm = _TM if m % _TM == 0 else m
    num_mt = m // tm
    row_spec = pl.BlockSpec((n, d), lambda: (0, 0))
    neg_spec = pl.BlockSpec((m, d), lambda: (0, 0))
    body = functools.partial(_mono_kernel, n=n, tm=tm, num_mt=num_mt)
    return pl.pallas_call(
        body,
        grid=(),
        in_specs=[row_spec, row_spec, neg_spec, row_spec, row_spec, neg_spec],
        out_specs=pl.BlockSpec(memory_space=pltpu.MemorySpace.SMEM),
        out_shape=jax.ShapeDtypeStruct((2, 3), jnp.float32),
        compiler_params=pltpu.CompilerParams(),
    )(ref1, pos1, neg1, ref2, pos2, neg2)
